# baseline (speedup 1.0000x reference)
import functools

import jax
import jax.numpy as jnp
from jax import lax
from jax.experimental import pallas as pl
from jax.experimental.pallas import tpu as pltpu
from jax.experimental.pallas import tpu_sc as plsc

_NUM_CORES = 2
_NUM_SUBCORES = 16
_NUM_WORKERS = _NUM_CORES * _NUM_SUBCORES

_LANES = 16
_BB = 8
_LPAD = 128


def _matmul_body(t_ref, w_ref, b_ref, o_ref):
    y = (
        lax.dot_general(
            t_ref[...], w_ref[...], (((1,), (1,)), ((), ())),
            preferred_element_type=jnp.float32,
        )
        + b_ref[...]
    )
    h = y.shape[0] // 2
    o_ref[...] = jnp.concatenate([y[:h, :], y[h:, :]], axis=1)


def _transform_table(table, w, bias, blk):
    v, d = table.shape
    return pl.pallas_call(
        _matmul_body,
        grid=(v // blk,),
        in_specs=[
            pl.BlockSpec((blk, d), lambda i: (i, 0)),
            pl.BlockSpec((d, d), lambda i: (0, 0)),
            pl.BlockSpec((1, d), lambda i: (0, 0)),
        ],
        out_specs=pl.BlockSpec((blk // 2, 2 * d), lambda i: (i, 0)),
        out_shape=jax.ShapeDtypeStruct((v // 2, 2 * d), jnp.float32),
    )(table, w, bias.reshape(1, d))


def _make_gather(bsz, seq, d, half_blk):
    assert half_blk == 4000
    assert bsz % (_NUM_WORKERS * _BB) == 0
    per_w = bsz // _NUM_WORKERS
    n_slabs = per_w // _BB
    mesh = plsc.VectorSubcoreMesh(
        core_axis_name="c", subcore_axis_name="s",
        num_cores=_NUM_CORES, num_subcores=_NUM_SUBCORES,
    )

    @functools.partial(
        pl.kernel,
        mesh=mesh,
        out_type=jax.ShapeDtypeStruct((bsz, seq, d), jnp.float32),
        scratch_types=[
            pltpu.VMEM((_BB, _LPAD), jnp.int32),
            pltpu.VMEM((_BB, _LPAD), jnp.int32),
            [pltpu.VMEM((seq, 2 * d), jnp.float32) for _ in range(_BB)],
            pltpu.VMEM((_BB, seq, d), jnp.float32),
            pltpu.SemaphoreType.DMA,
        ],
        compiler_params=pltpu.CompilerParams(use_tc_tiling_on_sc=True),
    )
    def gather_kernel(ttab_hbm, xpad_hbm, out_hbm, idx_v, idx_g, rows,
                      slab_v, sem):
        wid = lax.axis_index("s") * _NUM_CORES + lax.axis_index("c")
        e0 = wid * per_w

        def slab(s, carry):
            b0 = pl.multiple_of(e0 + s * _BB, _BB)
            pltpu.sync_copy(xpad_hbm.at[pl.ds(b0, _BB)], idx_v)
            for j in range(_BB):
                for k in range(_LPAD // _LANES):
                    sl = pl.ds(k * _LANES, _LANES)
                    v_ = idx_v[j, sl]
                    u = jax.lax.shift_right_logical(
                        jax.lax.shift_right_logical(v_, 5) * 33555, 22
                    )
                    idx_g[j, sl] = (
                        jax.lax.shift_right_logical(u, 1) * half_blk
                        + (v_ - u * half_blk)
                    )
                    idx_v[j, sl] = (u & 1) * d
            copies = []
            for j in range(_BB):
                copies.append(
                    pltpu.async_copy(
                        ttab_hbm.at[idx_g.at[j, pl.ds(0, seq)]],
                        rows[j],
                        sem,
                    )
                )
            for j in range(_BB):
                copies[j].wait()

                def compact(t, c, j=j):
                    iv = idx_v[j, pl.ds(t, _LANES)]
                    half = iv[0]
                    for k in range(d // _LANES):
                        slab_v[j, t, pl.ds(k * _LANES, _LANES)] = (
                            rows[j][t, pl.ds(half + k * _LANES, _LANES)]
                        )
                    return c

                lax.fori_loop(0, seq, compact, 0, unroll=False)
            pltpu.sync_copy(slab_v, out_hbm.at[pl.ds(b0, _BB)])
            return carry

        lax.fori_loop(0, n_slabs, slab, 0, unroll=False)

    return gather_kernel


def kernel(x, table, W, b):
    v, d = table.shape
    bsz, seq = x.shape

    blk = 8000
    ttable = _transform_table(table, W, b, blk=blk)

    xpad = jnp.pad(x, ((0, 0), (0, _LPAD - seq)))
    return _make_gather(bsz, seq, d, blk // 2)(ttable, xpad)


# speedup vs baseline: 1.3449x; 1.3449x over previous
import functools

import jax
import jax.numpy as jnp
from jax import lax
from jax.experimental import pallas as pl
from jax.experimental.pallas import tpu as pltpu
from jax.experimental.pallas import tpu_sc as plsc

_NUM_CORES = 2
_NUM_SUBCORES = 16
_NUM_WORKERS = _NUM_CORES * _NUM_SUBCORES

_LANES = 16
_BB = 8
_LPAD = 128


def _matmul_body(t_ref, w_ref, b_ref, o_ref):
    y = (
        lax.dot_general(
            t_ref[...], w_ref[...], (((1,), (1,)), ((), ())),
            preferred_element_type=jnp.float32,
        )
        + b_ref[...]
    )
    h = y.shape[0] // 2
    o_ref[...] = jnp.concatenate([y[:h, :], y[h:, :]], axis=1)


def _transform_table(table, w, bias, blk):
    v, d = table.shape
    return pl.pallas_call(
        _matmul_body,
        grid=(v // blk,),
        in_specs=[
            pl.BlockSpec((blk, d), lambda i: (i, 0)),
            pl.BlockSpec((d, d), lambda i: (0, 0)),
            pl.BlockSpec((1, d), lambda i: (0, 0)),
        ],
        out_specs=pl.BlockSpec((blk // 2, 2 * d), lambda i: (i, 0)),
        out_shape=jax.ShapeDtypeStruct((v // 2, 2 * d), jnp.float32),
    )(table, w, bias.reshape(1, d))


def _make_gather(bsz, seq, d, half_blk):
    assert half_blk == 4000
    assert bsz % (_NUM_WORKERS * _BB) == 0
    per_w = bsz // _NUM_WORKERS
    n_slabs = per_w // _BB
    mesh = plsc.VectorSubcoreMesh(
        core_axis_name="c", subcore_axis_name="s",
        num_cores=_NUM_CORES, num_subcores=_NUM_SUBCORES,
    )

    @functools.partial(
        pl.kernel,
        mesh=mesh,
        out_type=jax.ShapeDtypeStruct((bsz, seq, d), jnp.float32),
        scratch_types=[
            pltpu.VMEM((_BB, _LPAD), jnp.int32),
            pltpu.VMEM((_BB, _LPAD), jnp.int32),
            [pltpu.VMEM((seq, 2 * d), jnp.float32) for _ in range(_BB)],
            pltpu.VMEM((_BB, seq, d), jnp.float32),
            pltpu.SemaphoreType.DMA,
        ],
        compiler_params=pltpu.CompilerParams(use_tc_tiling_on_sc=True),
    )
    def gather_kernel(ttab_hbm, xpad_hbm, out_hbm, idx_v, idx_g, rows,
                      slab_v, sem):
        wid = lax.axis_index("s") * _NUM_CORES + lax.axis_index("c")
        e0 = wid * per_w

        def slab(s, carry):
            b0 = pl.multiple_of(e0 + s * _BB, _BB)
            pltpu.sync_copy(xpad_hbm.at[pl.ds(b0, _BB)], idx_v)
            for j in range(_BB):
                for k in range(_LPAD // _LANES):
                    sl = pl.ds(k * _LANES, _LANES)
                    v_ = idx_v[j, sl]
                    u = jax.lax.shift_right_logical(
                        jax.lax.shift_right_logical(v_, 5) * 33555, 22
                    )
                    idx_g[j, sl] = (
                        jax.lax.shift_right_logical(u, 1) * half_blk
                        + (v_ - u * half_blk)
                    )
                    idx_v[j, sl] = u & 1
            copies = []
            for j in range(_BB):
                copies.append(
                    pltpu.async_copy(
                        ttab_hbm.at[idx_g.at[j, pl.ds(0, seq)]],
                        rows[j],
                        sem,
                    )
                )
            for j in range(_BB):
                copies[j].wait()

                zero16 = jnp.zeros((_LANES, 1), jnp.int32)
                dnums = lax.GatherDimensionNumbers(
                    offset_dims=(), collapsed_slice_dims=(0,),
                    start_index_map=(0,),
                )

                def compact(t, c, j=j):
                    iv = idx_v[j, pl.ds(t, _LANES)]
                    flag = lax.gather(
                        iv, zero16, dnums, (1,),
                        mode=lax.GatherScatterMode.PROMISE_IN_BOUNDS,
                    ).astype(jnp.float32)
                    for k in range(d // _LANES):
                        lo = rows[j][t, pl.ds(k * _LANES, _LANES)]
                        hi = rows[j][t, pl.ds(d + k * _LANES, _LANES)]
                        slab_v[j, t, pl.ds(k * _LANES, _LANES)] = (
                            lo + flag * (hi - lo)
                        )
                    return c

                lax.fori_loop(0, seq, compact, 0, unroll=False)
            pltpu.sync_copy(slab_v, out_hbm.at[pl.ds(b0, _BB)])
            return carry

        lax.fori_loop(0, n_slabs, slab, 0, unroll=False)

    return gather_kernel


def kernel(x, table, W, b):
    v, d = table.shape
    bsz, seq = x.shape

    blk = 8000
    ttable = _transform_table(table, W, b, blk=blk)

    xpad = jnp.pad(x, ((0, 0), (0, _LPAD - seq)))
    return _make_gather(bsz, seq, d, blk // 2)(ttable, xpad)
